# baseline (speedup 1.0000x reference)
import functools

import jax
import jax.numpy as jnp
from jax import lax
from jax.experimental import pallas as pl
from jax.experimental.pallas import tpu as pltpu
from jax.experimental.pallas import tpu_sc as plsc

B = 16384
C = 64
NC = 2
NS = 16
NW = NC * NS
BPW = B // NW
CHUNK = 128
NCH = BPW // CHUNK
L = 16


def _vsqrt(x):
    i = lax.bitcast_convert_type(x, jnp.int32)
    i = (i >> 1) + jnp.int32(0x1FBD1DF5)
    y = lax.bitcast_convert_type(i, jnp.float32)
    y = 0.5 * (y + x / y)
    y = 0.5 * (y + x / y)
    y = 0.5 * (y + x / y)
    return y


def _lanesum(v):
    acc = v[0]
    for i in range(1, L):
        acc = acc + v[i]
    return acc


def _body(user_hbm, item_hbm, ustruct_hbm, istruct_hbm, rh_hbm, rel_hbm,
          out_hbm, uidx_v, iidx_v, urows_v, irows_v, rh_v, rel_v, out_v,
          gsem):
    wid = lax.axis_index("s") * NC + lax.axis_index("c")
    base = wid * BPW

    for j in range(NCH):
        pltpu.sync_copy(user_hbm.at[pl.ds(base + j * CHUNK, CHUNK)],
                        uidx_v.at[j])
        pltpu.sync_copy(item_hbm.at[pl.ds(base + j * CHUNK, CHUNK)],
                        iidx_v.at[j])
    pltpu.sync_copy(rh_hbm, rh_v)
    pltpu.sync_copy(rel_hbm, rel_v)

    copies = []
    for j in range(NCH):
        copies.append(pltpu.async_copy(
            ustruct_hbm.at[uidx_v.at[j]],
            urows_v.at[pl.ds(j * CHUNK, CHUNK)], gsem))
        copies.append(pltpu.async_copy(
            istruct_hbm.at[iidx_v.at[j]],
            irows_v.at[pl.ds(j * CHUNK, CHUNK)], gsem))
    for cp in copies:
        cp.wait()

    rh = [rh_v[pl.ds(k * L, L)] for k in range(C // L)]
    rele = [rel_v[pl.ds(k * L, L)] + 1e-6 for k in range(C // L)]
    s = rh[0] * rh[0]
    p = rh[0] * rele[0]
    for k in range(1, C // L):
        s = s + rh[k] * rh[k]
        p = p + rh[k] * rele[k]
    n2 = _lanesum(s)
    n2v = jnp.full((L,), 0.0, jnp.float32) + n2
    invv = 1.0 / jnp.maximum(_vsqrt(n2v), 1e-12)
    inv = invv[0]
    g2 = n2 * inv * inv
    rho = _lanesum(p) * inv
    ca = 2.0 - g2
    cb = 2.0 * rho
    rhn = [r * inv for r in rh]

    def group_body(g, _):
        rvec = g * L + lax.iota(jnp.int32, L)
        acc_a = jnp.zeros((L,), jnp.float32)
        acc_d = jnp.zeros((L,), jnp.float32)
        for c in range(C):
            cvec = jnp.full((L,), c, jnp.int32)
            u = plsc.load_gather(urows_v, [rvec, cvec])
            it = plsc.load_gather(irows_v, [rvec, cvec])
            d = u - it
            e = d + rele[c // L][c % L]
            acc_a = acc_a + e * e
            acc_d = acc_d + d * rhn[c // L][c % L]
        ssq = acc_a - ca * acc_d * acc_d - cb * acc_d
        out_v[pl.ds(g * L, L)] = _vsqrt(ssq)
        return 0

    lax.fori_loop(0, BPW // L, group_body, 0)

    pltpu.sync_copy(out_v, out_hbm.at[pl.ds(base, BPW)])


@jax.jit
def _transh(user, item, user_structure, item_structure, rh, rel):
    mesh = plsc.VectorSubcoreMesh(core_axis_name="c", subcore_axis_name="s")
    return pl.kernel(
        _body,
        out_type=jax.ShapeDtypeStruct((B,), jnp.float32),
        mesh=mesh,
        compiler_params=pltpu.CompilerParams(needs_layout_passes=False,
                                             use_tc_tiling_on_sc=False),
        scratch_types=[
            pltpu.VMEM((NCH, CHUNK), jnp.int32),
            pltpu.VMEM((NCH, CHUNK), jnp.int32),
            pltpu.VMEM((BPW, C), jnp.float32),
            pltpu.VMEM((BPW, C), jnp.float32),
            pltpu.VMEM((C,), jnp.float32),
            pltpu.VMEM((C,), jnp.float32),
            pltpu.VMEM((BPW,), jnp.float32),
            pltpu.SemaphoreType.DMA,
        ],
    )(user, item, user_structure, item_structure, rh, rel)


def kernel(user, item, user_structure, item_structure, relation_embedding,
           relationHyper):
    rh = relationHyper.reshape(C)
    rel = relation_embedding.reshape(C)
    return _transh(user.astype(jnp.int32), item.astype(jnp.int32),
                   user_structure, item_structure, rh, rel)
